# baseline (speedup 1.0000x reference)
import jax
import jax.numpy as jnp
import numpy as np
from jax import lax
from jax.experimental import pallas as pl
from jax.experimental.pallas import tpu as pltpu

D_MIN, D_MAX_RBF, N_BASES = 0.0, 4.5, 16
RADIUS = 5.0
G = 64
CHUNK = 128

_OFFSETS = np.linspace(D_MIN, D_MAX_RBF, N_BASES).astype(np.float32)
_COEFF = np.float32(-0.5 / (_OFFSETS[1] - _OFFSETS[0]) ** 2)


def _cfconv_body(lo_ref, nc_ref, feats_ref, csrc_ref, meta_ref,
                 w1_ref, w2_ref, out_ref):
    g = pl.program_id(0)
    lo = lo_ref[g]
    nc = nc_ref[g]

    meta = meta_ref[0]
    cdx = meta[:, 0:1]
    cdy = meta[:, 1:2]
    cdz = meta[:, 2:3]
    bd = meta[:, 3:4]
    gd = meta[:, 4:5]

    step = np.float32((D_MAX_RBF - D_MIN) / (N_BASES - 1))
    offc = D_MIN + step * lax.broadcasted_iota(
        jnp.int32, (N_BASES, 1), 0).astype(jnp.float32)
    w1 = w1_ref[...]
    w2 = w2_ref[...]

    def chunk_body(c, acc):
        s0 = pl.multiple_of(lo + c * CHUNK, 16)
        xs = feats_ref[pl.ds(s0, CHUNK), :]
        cs = csrc_ref[pl.ds(s0, CHUNK), :]
        csT = jnp.transpose(cs)
        sx = csT[0:1, :]
        sy = csT[1:2, :]
        sz = csT[2:3, :]
        sb = csT[3:4, :]

        ddx = cdx - sx
        ddy = cdy - sy
        ddz = cdz - sz
        d2 = ddx * ddx + ddy * ddy + ddz * ddz
        dT = jnp.sqrt(d2)

        sidx = s0.astype(jnp.float32) + lax.broadcasted_iota(
            jnp.int32, (1, CHUNK), 1).astype(jnp.float32)
        maskT = ((sb == bd) & (sidx != gd)
                 & (d2 <= RADIUS * RADIUS)).astype(jnp.float32)

        ones_row = jnp.full((1, CHUNK), 1.0, jnp.bfloat16)
        rows = []
        for j in range(G):
            rbf_t = maskT[j:j + 1, :] * jnp.exp(
                _COEFF * (dT[j:j + 1, :] - offc) ** 2)
            h = lax.dot_general(
                rbf_t.astype(jnp.bfloat16), w1, (((0,), (0,)), ((), ())),
                preferred_element_type=jnp.float32)
            hb = jax.nn.relu(h.astype(jnp.bfloat16))
            z = jnp.dot(hb, w2, preferred_element_type=jnp.float32)
            mb = jax.nn.relu(z.astype(jnp.bfloat16))
            p = xs * mb
            rows.append(lax.dot_general(
                ones_row, p, (((1,), (0,)), ((), ())),
                preferred_element_type=jnp.float32))
        return acc + jnp.concatenate(rows, axis=0)

    acc = jnp.zeros((G, 128), dtype=jnp.float32)
    out_ref[...] = lax.fori_loop(0, nc, chunk_body, acc)


@jax.jit
def kernel(node_feats, coords, batch_index, W1, W2):
    V, H = node_feats.shape
    b = batch_index.astype(jnp.int32)
    bf = b.astype(jnp.float32)

    VP = V + 2 * CHUNK
    pad = VP - V
    feats_p = jnp.pad(node_feats.astype(jnp.bfloat16), ((0, pad), (0, 0)))
    coords_p = jnp.pad(coords, ((0, pad), (0, 0)))
    bf_p = jnp.pad(bf, (0, pad), constant_values=-7.0)
    csrc_p = jnp.concatenate(
        [coords_p, bf_p[:, None], jnp.zeros((VP, 4), jnp.float32)], axis=1)

    num_groups = (V + G - 1) // G
    VG = num_groups * G
    dpad = VG - V
    gidx = jnp.arange(V, dtype=jnp.float32)
    zeros = jnp.zeros((V,), jnp.float32)
    meta = jnp.stack([coords[:, 0], coords[:, 1], coords[:, 2],
                      bf, gidx, zeros, zeros, zeros], axis=1)
    dfill = jnp.tile(jnp.asarray([0, 0, 0, -9, -1, 0, 0, 0], jnp.float32),
                     (dpad, 1))
    meta = jnp.concatenate([meta, dfill], axis=0).reshape(num_groups, G, 8)

    bq = jnp.concatenate([b, jnp.full((dpad,), b[-1], jnp.int32)])
    br = bq.reshape(num_groups, G)
    lo = jnp.searchsorted(b, br[:, 0], side='left').astype(jnp.int32)
    hi = jnp.searchsorted(b, br[:, G - 1], side='right').astype(jnp.int32)
    lo8 = (lo // 16) * 16
    nchunks = ((hi - lo8 + CHUNK - 1) // CHUNK).astype(jnp.int32)

    grid_spec = pltpu.PrefetchScalarGridSpec(
        num_scalar_prefetch=2,
        grid=(num_groups,),
        in_specs=[
            pl.BlockSpec((VP, H), lambda g, *_: (0, 0)),
            pl.BlockSpec((VP, 8), lambda g, *_: (0, 0)),
            pl.BlockSpec((1, G, 8), lambda g, *_: (g, 0, 0)),
            pl.BlockSpec((N_BASES, H), lambda g, *_: (0, 0)),
            pl.BlockSpec((H, H), lambda g, *_: (0, 0)),
        ],
        out_specs=pl.BlockSpec((G, H), lambda g, *_: (g, 0)),
    )

    out = pl.pallas_call(
        _cfconv_body,
        grid_spec=grid_spec,
        out_shape=jax.ShapeDtypeStruct((VG, H), jnp.float32),
    )(lo8, nchunks, feats_p, csrc_p, meta,
      W1.astype(jnp.bfloat16), W2.astype(jnp.bfloat16))
    return out[:V]


# speedup vs baseline: 6.4941x; 6.4941x over previous
import jax
import jax.numpy as jnp
import numpy as np
from jax import lax
from jax.experimental import pallas as pl
from jax.experimental.pallas import tpu as pltpu

D_MIN, D_MAX_RBF, N_BASES = 0.0, 4.5, 16
RADIUS = 5.0
G = 64
CHUNK = 128

_OFFSETS = np.linspace(D_MIN, D_MAX_RBF, N_BASES).astype(np.float32)
_COEFF = np.float32(-0.5 / (_OFFSETS[1] - _OFFSETS[0]) ** 2)


def _cfconv_body(lo_ref, nc_ref, feats_ref, csrc_ref, meta_ref,
                 w1_ref, w2_ref, out_ref):
    g = pl.program_id(0)
    lo = lo_ref[g]
    nc = nc_ref[g]

    meta = meta_ref[0]
    cdx = meta[:, 0:1]
    cdy = meta[:, 1:2]
    cdz = meta[:, 2:3]
    bd = meta[:, 3:4]
    gd = meta[:, 4:5]

    step = np.float32((D_MAX_RBF - D_MIN) / (N_BASES - 1))
    offc = D_MIN + step * lax.broadcasted_iota(
        jnp.int32, (N_BASES, 1), 0).astype(jnp.float32)
    w1 = w1_ref[...]
    w2 = w2_ref[...]

    def chunk_body(c, acc):
        s0 = pl.multiple_of(lo + c * CHUNK, 16)
        xs = feats_ref[pl.ds(s0, CHUNK), :]
        cs = csrc_ref[pl.ds(s0, CHUNK), :]
        csT = jnp.transpose(cs)
        sx = csT[0:1, :]
        sy = csT[1:2, :]
        sz = csT[2:3, :]
        sb = csT[3:4, :]

        ddx = cdx - sx
        ddy = cdy - sy
        ddz = cdz - sz
        d2 = ddx * ddx + ddy * ddy + ddz * ddz
        dT = jnp.sqrt(d2)

        sidx = s0.astype(jnp.float32) + lax.broadcasted_iota(
            jnp.int32, (1, CHUNK), 1).astype(jnp.float32)
        maskT = ((sb == bd) & (sidx != gd)
                 & (d2 <= RADIUS * RADIUS)).astype(jnp.float32)

        rows = []
        for j in range(G):
            rbf_t = maskT[j:j + 1, :] * jnp.exp(
                _COEFF * (dT[j:j + 1, :] - offc) ** 2)
            h = jax.nn.relu(lax.dot_general(
                rbf_t.astype(jnp.bfloat16), w1, (((0,), (0,)), ((), ())),
                preferred_element_type=jnp.float32))
            m = jax.nn.relu(
                jnp.dot(h.astype(jnp.bfloat16), w2,
                        preferred_element_type=jnp.float32))
            rows.append(jnp.sum(xs * m, axis=0, keepdims=True))
        return acc + jnp.concatenate(rows, axis=0)

    acc = jnp.zeros((G, 128), dtype=jnp.float32)
    out_ref[...] = lax.fori_loop(0, nc, chunk_body, acc)


@jax.jit
def kernel(node_feats, coords, batch_index, W1, W2):
    V, H = node_feats.shape
    b = batch_index.astype(jnp.int32)
    bf = b.astype(jnp.float32)

    VP = V + 2 * CHUNK
    pad = VP - V
    feats_p = jnp.pad(node_feats, ((0, pad), (0, 0)))
    coords_p = jnp.pad(coords, ((0, pad), (0, 0)))
    bf_p = jnp.pad(bf, (0, pad), constant_values=-7.0)
    csrc_p = jnp.concatenate(
        [coords_p, bf_p[:, None], jnp.zeros((VP, 4), jnp.float32)], axis=1)

    num_groups = (V + G - 1) // G
    VG = num_groups * G
    dpad = VG - V
    gidx = jnp.arange(V, dtype=jnp.float32)
    zeros = jnp.zeros((V,), jnp.float32)
    meta = jnp.stack([coords[:, 0], coords[:, 1], coords[:, 2],
                      bf, gidx, zeros, zeros, zeros], axis=1)
    dfill = jnp.tile(jnp.asarray([0, 0, 0, -9, -1, 0, 0, 0], jnp.float32),
                     (dpad, 1))
    meta = jnp.concatenate([meta, dfill], axis=0).reshape(num_groups, G, 8)

    bq = jnp.concatenate([b, jnp.full((dpad,), b[-1], jnp.int32)])
    br = bq.reshape(num_groups, G)
    lo = jnp.searchsorted(b, br[:, 0], side='left').astype(jnp.int32)
    hi = jnp.searchsorted(b, br[:, G - 1], side='right').astype(jnp.int32)
    lo8 = (lo // 16) * 16
    nchunks = ((hi - lo8 + CHUNK - 1) // CHUNK).astype(jnp.int32)

    grid_spec = pltpu.PrefetchScalarGridSpec(
        num_scalar_prefetch=2,
        grid=(num_groups,),
        in_specs=[
            pl.BlockSpec((VP, H), lambda g, *_: (0, 0)),
            pl.BlockSpec((VP, 8), lambda g, *_: (0, 0)),
            pl.BlockSpec((1, G, 8), lambda g, *_: (g, 0, 0)),
            pl.BlockSpec((N_BASES, H), lambda g, *_: (0, 0)),
            pl.BlockSpec((H, H), lambda g, *_: (0, 0)),
        ],
        out_specs=pl.BlockSpec((G, H), lambda g, *_: (g, 0)),
    )

    out = pl.pallas_call(
        _cfconv_body,
        grid_spec=grid_spec,
        out_shape=jax.ShapeDtypeStruct((VG, H), jnp.float32),
    )(lo8, nchunks, feats_p, csrc_p, meta,
      W1.astype(jnp.bfloat16), W2.astype(jnp.bfloat16))
    return out[:V]


# speedup vs baseline: 6.6594x; 1.0255x over previous
import jax
import jax.numpy as jnp
import numpy as np
from jax import lax
from jax.experimental import pallas as pl
from jax.experimental.pallas import tpu as pltpu

D_MIN, D_MAX_RBF, N_BASES = 0.0, 4.5, 16
RADIUS = 5.0
G = 64
CHUNK = 128

_OFFSETS = np.linspace(D_MIN, D_MAX_RBF, N_BASES).astype(np.float32)
_COEFF = np.float32(-0.5 / (_OFFSETS[1] - _OFFSETS[0]) ** 2)


def _cfconv_body(lo_ref, nc_ref, feats_ref, csrc_ref, meta_ref,
                 w1_ref, w2_ref, out_ref):
    g = pl.program_id(0)
    lo = lo_ref[g]
    nc = nc_ref[g]

    meta = meta_ref[0]
    cdx = meta[:, 0:1]
    cdy = meta[:, 1:2]
    cdz = meta[:, 2:3]
    bd = meta[:, 3:4]
    gd = meta[:, 4:5]

    step = np.float32((D_MAX_RBF - D_MIN) / (N_BASES - 1))
    offc = D_MIN + step * lax.broadcasted_iota(
        jnp.int32, (N_BASES, 1), 0).astype(jnp.float32)
    w1 = w1_ref[...]
    w2 = w2_ref[...]

    def chunk_body(c, acc):
        s0 = pl.multiple_of(lo + c * CHUNK, 8)
        xs = feats_ref[pl.ds(s0, CHUNK), :]
        cs = csrc_ref[pl.ds(s0, CHUNK), :]
        csT = jnp.transpose(cs)
        sx = csT[0:1, :]
        sy = csT[1:2, :]
        sz = csT[2:3, :]
        sb = csT[3:4, :]

        ddx = cdx - sx
        ddy = cdy - sy
        ddz = cdz - sz
        d2 = ddx * ddx + ddy * ddy + ddz * ddz
        dT = jnp.sqrt(d2)

        sidx = s0.astype(jnp.float32) + lax.broadcasted_iota(
            jnp.int32, (1, CHUNK), 1).astype(jnp.float32)
        maskT = ((sb == bd) & (sidx != gd)
                 & (d2 <= RADIUS * RADIUS)).astype(jnp.float32)

        rows = []
        for j in range(G):
            rbf_t = maskT[j:j + 1, :] * jnp.exp(
                _COEFF * (dT[j:j + 1, :] - offc) ** 2)
            h = jax.nn.relu(lax.dot_general(
                rbf_t.astype(jnp.bfloat16), w1, (((0,), (0,)), ((), ())),
                preferred_element_type=jnp.float32))
            m = jax.nn.relu(
                jnp.dot(h.astype(jnp.bfloat16), w2,
                        preferred_element_type=jnp.float32))
            rows.append(jnp.sum(xs * m, axis=0, keepdims=True))
        return acc + jnp.concatenate(rows, axis=0)

    acc = jnp.zeros((G, 128), dtype=jnp.float32)
    out_ref[...] = lax.fori_loop(0, nc, chunk_body, acc)


@jax.jit
def kernel(node_feats, coords, batch_index, W1, W2):
    V, H = node_feats.shape
    b = batch_index.astype(jnp.int32)
    bf = b.astype(jnp.float32)

    VP = V + 2 * CHUNK
    pad = VP - V
    feats_p = jnp.pad(node_feats, ((0, pad), (0, 0)))
    coords_p = jnp.pad(coords, ((0, pad), (0, 0)))
    bf_p = jnp.pad(bf, (0, pad), constant_values=-7.0)
    csrc_p = jnp.concatenate(
        [coords_p, bf_p[:, None], jnp.zeros((VP, 4), jnp.float32)], axis=1)

    num_groups = (V + G - 1) // G
    VG = num_groups * G
    dpad = VG - V
    gidx = jnp.arange(V, dtype=jnp.float32)
    zeros = jnp.zeros((V,), jnp.float32)
    meta = jnp.stack([coords[:, 0], coords[:, 1], coords[:, 2],
                      bf, gidx, zeros, zeros, zeros], axis=1)
    dfill = jnp.tile(jnp.asarray([0, 0, 0, -9, -1, 0, 0, 0], jnp.float32),
                     (dpad, 1))
    meta = jnp.concatenate([meta, dfill], axis=0).reshape(num_groups, G, 8)

    bq = jnp.concatenate([b, jnp.full((dpad,), b[-1], jnp.int32)])
    br = bq.reshape(num_groups, G)
    lo = jnp.searchsorted(b, br[:, 0], side='left').astype(jnp.int32)
    hi = jnp.searchsorted(b, br[:, G - 1], side='right').astype(jnp.int32)
    lo8 = (lo // 8) * 8
    nchunks = ((hi - lo8 + CHUNK - 1) // CHUNK).astype(jnp.int32)

    grid_spec = pltpu.PrefetchScalarGridSpec(
        num_scalar_prefetch=2,
        grid=(num_groups,),
        in_specs=[
            pl.BlockSpec((VP, H), lambda g, *_: (0, 0)),
            pl.BlockSpec((VP, 8), lambda g, *_: (0, 0)),
            pl.BlockSpec((1, G, 8), lambda g, *_: (g, 0, 0)),
            pl.BlockSpec((N_BASES, H), lambda g, *_: (0, 0)),
            pl.BlockSpec((H, H), lambda g, *_: (0, 0)),
        ],
        out_specs=pl.BlockSpec((G, H), lambda g, *_: (g, 0)),
    )

    out = pl.pallas_call(
        _cfconv_body,
        grid_spec=grid_spec,
        out_shape=jax.ShapeDtypeStruct((VG, H), jnp.float32),
    )(lo8, nchunks, feats_p, csrc_p, meta,
      W1.astype(jnp.bfloat16), W2.astype(jnp.bfloat16))
    return out[:V]
